# baseline (speedup 1.0000x reference)
import functools
import math

import jax
import jax.numpy as jnp
from jax import lax
from jax.experimental import pallas as pl
from jax.experimental.pallas import tpu as pltpu
from jax.experimental.pallas import tpu_sc as plsc

D_MODEL = 64
D_PAD = 128
SCALE = math.sqrt(D_MODEL)

NUM_CORES = 2
NUM_SUBCORES = 16
NUM_WORKERS = NUM_CORES * NUM_SUBCORES

SEQ = 200
SBLK = 128

PAD_BLK = 2048


def _pad_body(t_ref, o_ref):
    o_ref[:, 0:D_MODEL] = t_ref[...].T


def _widen_table(table_t):
    vocab = table_t.shape[1]
    return pl.pallas_call(
        _pad_body,
        grid=(pl.cdiv(vocab, PAD_BLK),),
        in_specs=[pl.BlockSpec((D_MODEL, PAD_BLK), lambda i: (0, i))],
        out_specs=pl.BlockSpec((PAD_BLK, D_PAD), lambda i: (i, 0)),
        out_shape=jax.ShapeDtypeStruct((vocab, D_PAD), jnp.float32),
    )(table_t)


def _emb_body(xt_hbm, table_hbm, out_hbm, idx_v, g0, g1, t0, t1, gs0, gs1, ts0, ts1):
    wid = lax.axis_index("s") * NUM_CORES + lax.axis_index("c")
    scol = wid * SBLK

    pltpu.sync_copy(xt_hbm.at[:, pl.ds(scol, SBLK)], idx_v)

    def fire_gather(t, gbuf, gsem):
        pltpu.async_copy(table_hbm.at[idx_v.at[t]], gbuf, gsem)

    def wait_gather(t, gbuf, gsem):
        pltpu.make_async_copy(table_hbm.at[idx_v.at[t]], gbuf, gsem).wait()

    def out_ref(t):
        return out_hbm.at[t, :, pl.ds(scol, SBLK)]

    def tview(tbuf):
        return tbuf.at[:, pl.ds(0, SBLK)]

    fire_gather(0, g0, gs0)

    lanes = lax.iota(jnp.int32, 16)
    lane_ids = [lanes + (c * 16) for c in range(D_MODEL // 16)]

    def pair_body(p, carry):
        for b in (0, 1):
            t = 2 * p + b
            gbuf, gsem = (g0, gs0) if b == 0 else (g1, gs1)
            nbuf, nsem = (g1, gs1) if b == 0 else (g0, gs0)
            tbuf, tsem = (t0, ts0) if b == 0 else (t1, ts1)

            @pl.when(t + 1 < SEQ)
            def _():
                fire_gather(t + 1, nbuf, nsem)

            wait_gather(t, gbuf, gsem)

            @pl.when(t >= 2)
            def _():
                pltpu.make_async_copy(tview(tbuf), out_ref(t - 2), tsem).wait()

            def srow(s, c2):
                svec = jnp.full((16,), s, jnp.int32)
                for c in range(D_MODEL // 16):
                    v = gbuf[s, pl.ds(c * 16, 16)] * SCALE
                    plsc.store_scatter(tbuf, [lane_ids[c], svec], v)
                return c2

            lax.fori_loop(0, SBLK, srow, 0)

            pltpu.async_copy(tview(tbuf), out_ref(t), tsem)
        return carry

    lax.fori_loop(0, SEQ // 2, pair_body, 0)

    pltpu.make_async_copy(tview(t0), out_ref(SEQ - 2), ts0).wait()
    pltpu.make_async_copy(tview(t1), out_ref(SEQ - 1), ts1).wait()


@jax.jit
def kernel(x, table):
    n_sent, seq = x.shape
    assert seq == SEQ and n_sent == NUM_WORKERS * SBLK

    table_wide = _widen_table(table.T)
    xt = x.T

    mesh = plsc.VectorSubcoreMesh(core_axis_name="c", subcore_axis_name="s")
    out_t = pl.kernel(
        _emb_body,
        mesh=mesh,
        compiler_params=pltpu.CompilerParams(needs_layout_passes=False),
        out_type=jax.ShapeDtypeStruct((SEQ, D_MODEL, n_sent), jnp.float32),
        scratch_types=[
            pltpu.VMEM((SEQ, SBLK), jnp.int32),
            pltpu.VMEM((SBLK, D_PAD), jnp.float32),
            pltpu.VMEM((SBLK, D_PAD), jnp.float32),
            pltpu.VMEM((D_MODEL, SBLK + 1), jnp.float32),
            pltpu.VMEM((D_MODEL, SBLK + 1), jnp.float32),
            pltpu.SemaphoreType.DMA,
            pltpu.SemaphoreType.DMA,
            pltpu.SemaphoreType.DMA,
            pltpu.SemaphoreType.DMA,
        ],
    )(xt, table_wide)
    return out_t.transpose(2, 0, 1)


# speedup vs baseline: 1.3640x; 1.3640x over previous
import functools
import math

import jax
import jax.numpy as jnp
from jax import lax
from jax.experimental import pallas as pl
from jax.experimental.pallas import tpu as pltpu
from jax.experimental.pallas import tpu_sc as plsc

D_MODEL = 64
D_PAD = 128
SCALE = math.sqrt(D_MODEL)

NUM_CORES = 2
NUM_SUBCORES = 16
NUM_WORKERS = NUM_CORES * NUM_SUBCORES

SEQ = 200
SBLK = 128

PAD_BLK = 2048


def _pad_body(t_ref, o_ref):
    o_ref[:, 0:D_MODEL] = t_ref[...].T


def _widen_table(table_t):
    vocab = table_t.shape[1]
    return pl.pallas_call(
        _pad_body,
        grid=(pl.cdiv(vocab, PAD_BLK),),
        in_specs=[pl.BlockSpec((D_MODEL, PAD_BLK), lambda i: (0, i))],
        out_specs=pl.BlockSpec((PAD_BLK, D_PAD), lambda i: (i, 0)),
        out_shape=jax.ShapeDtypeStruct((vocab, D_PAD), jnp.float32),
    )(table_t)


def _emb_body(xt_hbm, table_hbm, out_hbm, idx_v, g0, g1, t0, t1, gs0, gs1, ts0, ts1):
    wid = lax.axis_index("s") * NUM_CORES + lax.axis_index("c")
    scol = wid * SBLK

    pltpu.sync_copy(xt_hbm.at[:, pl.ds(scol, SBLK)], idx_v)

    def fire_gather(t, gbuf, gsem):
        pltpu.async_copy(table_hbm.at[idx_v.at[t]], gbuf, gsem)

    def wait_gather(t, gbuf, gsem):
        pltpu.make_async_copy(table_hbm.at[idx_v.at[t]], gbuf, gsem).wait()

    def out_ref(t):
        return out_hbm.at[t, :, pl.ds(scol, SBLK)]

    def tview(tbuf):
        return tbuf.at[:, pl.ds(0, SBLK)]

    fire_gather(0, g0, gs0)

    lanes = lax.iota(jnp.int32, 16)
    lane_ids = [lanes + (c * 16) for c in range(D_MODEL // 16)]

    def pair_body(p, carry):
        for b in (0, 1):
            t = 2 * p + b
            gbuf, gsem = (g0, gs0) if b == 0 else (g1, gs1)
            nbuf, nsem = (g1, gs1) if b == 0 else (g0, gs0)
            tbuf, tsem = (t0, ts0) if b == 0 else (t1, ts1)

            @pl.when(t + 1 < SEQ)
            def _():
                fire_gather(t + 1, nbuf, nsem)

            wait_gather(t, gbuf, gsem)

            @pl.when(t >= 2)
            def _():
                pltpu.make_async_copy(tview(tbuf), out_ref(t - 2), tsem).wait()

            @plsc.parallel_loop(0, SBLK, unroll=4)
            def _srow(s):
                svec = jnp.full((16,), s, jnp.int32)
                for c in range(D_MODEL // 16):
                    v = gbuf[s, pl.ds(c * 16, 16)] * SCALE
                    plsc.store_scatter(tbuf, [lane_ids[c], svec], v)

            pltpu.async_copy(tview(tbuf), out_ref(t), tsem)
        return carry

    lax.fori_loop(0, SEQ // 2, pair_body, 0)

    pltpu.make_async_copy(tview(t0), out_ref(SEQ - 2), ts0).wait()
    pltpu.make_async_copy(tview(t1), out_ref(SEQ - 1), ts1).wait()


@jax.jit
def kernel(x, table):
    n_sent, seq = x.shape
    assert seq == SEQ and n_sent == NUM_WORKERS * SBLK

    table_wide = _widen_table(table.T)
    xt = x.T

    mesh = plsc.VectorSubcoreMesh(core_axis_name="c", subcore_axis_name="s")
    out_t = pl.kernel(
        _emb_body,
        mesh=mesh,
        compiler_params=pltpu.CompilerParams(needs_layout_passes=False),
        out_type=jax.ShapeDtypeStruct((SEQ, D_MODEL, n_sent), jnp.float32),
        scratch_types=[
            pltpu.VMEM((SEQ, SBLK), jnp.int32),
            pltpu.VMEM((SBLK, D_PAD), jnp.float32),
            pltpu.VMEM((SBLK, D_PAD), jnp.float32),
            pltpu.VMEM((D_MODEL, SBLK + 1), jnp.float32),
            pltpu.VMEM((D_MODEL, SBLK + 1), jnp.float32),
            pltpu.SemaphoreType.DMA,
            pltpu.SemaphoreType.DMA,
            pltpu.SemaphoreType.DMA,
            pltpu.SemaphoreType.DMA,
        ],
    )(xt, table_wide)
    return out_t.transpose(2, 0, 1)
